# baseline (speedup 1.0000x reference)
import functools

import jax
import jax.numpy as jnp
from jax import lax
from jax.experimental import pallas as pl
from jax.experimental.pallas import tpu as pltpu
from jax.experimental.pallas import tpu_sc as plsc

B, C, H, W = 8, 3, 152, 152
HW = H * W
CHW = C * HW
K = 50
KPAD = 64
NQ = 2
NCHUNK = CHW // NQ
COLLEN = NCHUNK // 16
NGROUPS = (COLLEN + 15) // 16
NSCAN = (NGROUPS + 15) // 16
NCH = 8
CLIP_LO = 0.0001
CLIP_HI = 1.0 - 0.0001


def _nms_body(x_ref, o_ref):
    x = x_ref[...]
    s = 1.0 / (1.0 + jnp.exp(-x))
    s = jnp.clip(s, CLIP_LO, CLIP_HI)
    negh = jnp.full((1, C, 1, W), -1.0, s.dtype)
    up = jnp.concatenate([s[:, :, 1:, :], negh], axis=2)
    dn = jnp.concatenate([negh, s[:, :, :-1, :]], axis=2)
    m = jnp.maximum(jnp.maximum(s, up), dn)
    negw = jnp.full((1, C, H, 1), -1.0, s.dtype)
    lf = jnp.concatenate([m[:, :, :, 1:], negw], axis=3)
    rt = jnp.concatenate([negw, m[:, :, :, :-1]], axis=3)
    m = jnp.maximum(jnp.maximum(m, lf), rt)
    o_ref[...] = jnp.where(m == s, s, 0.0)


_nms_call = pl.pallas_call(
    _nms_body,
    out_shape=jax.ShapeDtypeStruct((B, C, H, W), jnp.float32),
    grid=(B,),
    in_specs=[pl.BlockSpec((1, C, H, W), lambda i: (i, 0, 0, 0))],
    out_specs=pl.BlockSpec((1, C, H, W), lambda i: (i, 0, 0, 0)),
)


def _sigmoid_clip(x):
    s = 1.0 / (1.0 + jnp.exp(-x))
    return jnp.clip(s, CLIP_LO, CLIP_HI)


def _sc_body(scores_hbm, xy_hbm, rot_hbm, z_hbm, dim_hbm, out_hbm,
             arr, gm, lsc, lgi, shsc, shgi, csc, cgi, gidxb, gfeat, det, sem):
    c = lax.axis_index("c")
    s = lax.axis_index("s")
    b = c * (B // 2) + s // NQ
    q = s % NQ
    iota = lax.iota(jnp.int32, 16)
    neg1 = jnp.full((16,), -1.0, jnp.float32)
    big = jnp.full((16,), 1 << 30, jnp.int32)
    lane0 = iota == 0

    pltpu.sync_copy(scores_hbm.at[pl.ds(b * CHW + q * NCHUNK, NCHUNK)], arr)

    for v in range(KPAD // 16):
        lsc[pl.ds(v * 16, 16)] = neg1
        lgi[pl.ds(v * 16, 16)] = jnp.zeros((16,), jnp.int32)

    col_base = iota * COLLEN
    nchunk_m1 = jnp.full((16,), NCHUNK - 1, jnp.int32)
    collen_v = jnp.full((16,), COLLEN, jnp.int32)

    def grp_body(g, M):
        gb = col_base + g * 16
        acc = neg1
        for j in range(16):
            v = plsc.load_gather(arr, [gb + j])
            acc = jnp.maximum(acc, v)
        gm[pl.ds(g * 16, 16)] = acc
        return jnp.maximum(M, acc)

    M = lax.fori_loop(0, NGROUPS - 1, grp_body, neg1)
    gb = col_base + (NGROUPS - 1) * 16
    acc = neg1
    for j in range(COLLEN - (NGROUPS - 1) * 16):
        v = plsc.load_gather(arr, [gb + j])
        acc = jnp.maximum(acc, v)
    gm[pl.ds((NGROUPS - 1) * 16, 16)] = acc
    M = jnp.maximum(M, acc)

    def extract_body(i, M):
        m = jnp.max(M)
        mv = jnp.broadcast_to(m, (16,))
        l0v = plsc.all_reduce_ffs(M == mv)

        mink = big
        max2 = neg1
        cnt = jnp.zeros((16,), jnp.int32)
        one_i = jnp.full((16,), 1, jnp.int32)
        zero_i = jnp.zeros((16,), jnp.int32)
        for u in range(NSCAN):
            kk = iota + u * 16
            gidx = jnp.minimum(kk, NGROUPS - 1) * 16 + l0v
            v = plsc.load_gather(gm, [gidx])
            valid = kk < NGROUPS
            hit = (v == mv) & valid
            mink = jnp.minimum(mink, jnp.where(hit, kk, big))
            max2 = jnp.maximum(max2, jnp.where(hit | ~valid, neg1, v))
            cnt = cnt + jnp.where(hit, one_i, zero_i)
        k0 = jnp.min(mink)
        k0v = jnp.broadcast_to(k0, (16,))
        ndup = jnp.sum(cnt)

        tt = k0v * 16 + iota
        eidx = jnp.minimum(l0v * COLLEN + tt, nchunk_m1)
        v = plsc.load_gather(arr, [eidx])
        hit = (v == mv) & (tt < collen_v)
        j0v = plsc.all_reduce_ffs(hit)
        posv = l0v * COLLEN + k0v * 16 + j0v

        iv = jnp.broadcast_to(i, (16,))
        plsc.store_scatter(lsc, [iv], mv, mask=lane0)
        plsc.store_scatter(lgi, [iv], q * NCHUNK + posv, mask=lane0)

        plsc.store_scatter(arr, [posv], neg1, mask=lane0)
        v2 = plsc.load_gather(arr, [eidx])
        v2 = jnp.where(tt < collen_v, v2, neg1)
        gnv = jnp.broadcast_to(jnp.max(v2), (16,))
        plsc.store_scatter(gm, [k0v * 16 + l0v], gnv, mask=lane0)

        cmv = jnp.maximum(jnp.broadcast_to(jnp.max(max2), (16,)), gnv)
        cmv = jnp.where(jnp.broadcast_to(ndup >= 2, (16,)), mv, cmv)
        return jnp.where(iota == l0v, cmv, M)

    lax.fori_loop(0, K, extract_body, M)

    pltpu.sync_copy(lsc, shsc.at[pl.ds(s * KPAD, KPAD)])
    pltpu.sync_copy(lgi, shgi.at[pl.ds(s * KPAD, KPAD)])
    plsc.subcore_barrier()

    @pl.when(q == 0)
    def _leader():
        for j in range(NQ):
            pltpu.sync_copy(shsc.at[pl.ds((s + j) * KPAD, KPAD)],
                            csc.at[pl.ds(j * KPAD, KPAD)])
            pltpu.sync_copy(shgi.at[pl.ds((s + j) * KPAD, KPAD)],
                            cgi.at[pl.ds(j * KPAD, KPAD)])

        def merge_body(i, heads):
            hpos = iota * KPAD + heads
            v = plsc.load_gather(csc, [jnp.minimum(hpos, NQ * KPAD - 1)])
            v = jnp.where(iota < jnp.full((16,), NQ, jnp.int32), v, neg1)
            mx = jnp.max(v)
            mxv = jnp.broadcast_to(mx, (16,))
            j0 = jnp.min(jnp.where(v == mxv, iota, big))
            j0v = jnp.broadcast_to(j0, (16,))
            h0 = jnp.min(jnp.where(iota == j0v, heads, big))
            posv = j0v * KPAD + jnp.broadcast_to(h0, (16,))

            gv = plsc.load_gather(cgi, [posv])
            iv = jnp.broadcast_to(i, (16,))
            plsc.store_scatter(lsc, [iv], mxv, mask=lane0)
            plsc.store_scatter(lgi, [iv], gv, mask=lane0)
            return jnp.where(iota == j0v, heads + 1, heads)

        lax.fori_loop(0, K, merge_body, jnp.zeros((16,), jnp.int32))

        srcs = [(xy_hbm, 2, 0), (xy_hbm, 2, 1), (rot_hbm, 2, 0), (rot_hbm, 2, 1),
                (z_hbm, 1, 0), (dim_hbm, 3, 0), (dim_hbm, 3, 1), (dim_hbm, 3, 2)]
        for v in range(KPAD // 16):
            g_v = lgi[pl.ds(v * 16, 16)]
            hw_v = g_v % HW
            for ch, (_, nch, cin) in enumerate(srcs):
                gidxb[pl.ds(ch * KPAD + v * 16, 16)] = (b * nch + cin) * HW + hw_v

        copies = [pltpu.async_copy(ref.at[gidxb.at[pl.ds(ch * KPAD, KPAD)]],
                                   gfeat.at[pl.ds(ch * KPAD, KPAD)], sem)
                  for ch, (ref, _, _) in enumerate(srcs)]
        for cp in copies:
            cp.wait()

        for v in range(KPAD // 16):
            lanes = iota + v * 16
            ok = lanes < K
            sc_v = lsc[pl.ds(v * 16, 16)]
            g_v = lgi[pl.ds(v * 16, 16)]
            cls_v = (g_v // HW).astype(jnp.float32)
            hw_v = g_v % HW
            row_v = (hw_v // W).astype(jnp.float32)
            col_v = (hw_v % W).astype(jnp.float32)
            offx = _sigmoid_clip(gfeat[pl.ds(0 * KPAD + v * 16, 16)])
            offy = _sigmoid_clip(gfeat[pl.ds(1 * KPAD + v * 16, 16)])
            cols = [sc_v, col_v + offx, row_v + offy,
                    gfeat[pl.ds(4 * KPAD + v * 16, 16)],
                    gfeat[pl.ds(5 * KPAD + v * 16, 16)],
                    gfeat[pl.ds(6 * KPAD + v * 16, 16)],
                    gfeat[pl.ds(7 * KPAD + v * 16, 16)],
                    gfeat[pl.ds(2 * KPAD + v * 16, 16)],
                    gfeat[pl.ds(3 * KPAD + v * 16, 16)],
                    cls_v]
            for cc, val in enumerate(cols):
                plsc.store_scatter(det, [lanes * 10 + cc], val, mask=ok)

        pltpu.sync_copy(det, out_hbm.at[pl.ds(b * 512, 512)])


_sc_call = functools.partial(
    pl.kernel,
    out_type=jax.ShapeDtypeStruct((B * 512,), jnp.float32),
    mesh=plsc.VectorSubcoreMesh(core_axis_name="c", subcore_axis_name="s",
                                num_cores=1, num_subcores=16),
    compiler_params=pltpu.CompilerParams(needs_layout_passes=False),
    scratch_types=[
        pltpu.VMEM((NCHUNK,), jnp.float32),
        pltpu.VMEM((NGROUPS * 16,), jnp.float32),
        pltpu.VMEM((KPAD,), jnp.float32),
        pltpu.VMEM((KPAD,), jnp.int32),
        pltpu.VMEM_SHARED((16 * KPAD,), jnp.float32),
        pltpu.VMEM_SHARED((16 * KPAD,), jnp.int32),
        pltpu.VMEM((NQ * KPAD,), jnp.float32),
        pltpu.VMEM((NQ * KPAD,), jnp.int32),
        pltpu.VMEM((NCH * KPAD,), jnp.int32),
        pltpu.VMEM((NCH * KPAD,), jnp.float32),
        pltpu.VMEM((512,), jnp.float32),
        pltpu.SemaphoreType.DMA,
    ],
)(_sc_body)


def kernel(cls_preds, xy_preds, rot_preds, z_preds, dim_preds, K_arg):
    scores = _nms_call(cls_preds).reshape(B * CHW)
    det = _sc_call(scores, xy_preds.reshape(-1), rot_preds.reshape(-1),
                   z_preds.reshape(-1), dim_preds.reshape(-1))
    out = det.reshape(B, 512)[:, :K * 10].reshape(B, K, 10)
    one = jnp.asarray(K_arg // K_arg, dtype=out.dtype)
    return out * one


# speedup vs baseline: 1.0284x; 1.0284x over previous
import functools

import jax
import jax.numpy as jnp
from jax import lax
from jax.experimental import pallas as pl
from jax.experimental.pallas import tpu as pltpu
from jax.experimental.pallas import tpu_sc as plsc

B, C, H, W = 8, 3, 152, 152
HW = H * W
CHW = C * HW
K = 50
KPAD = 64
NQ = 4
NCHUNK = CHW // NQ
COLLEN = NCHUNK // 16
NGROUPS = (COLLEN + 15) // 16
NSCAN = (NGROUPS + 15) // 16
NCH = 8
CLIP_LO = 0.0001
CLIP_HI = 1.0 - 0.0001


def _nms_body(x_ref, o_ref):
    x = x_ref[...]
    s = 1.0 / (1.0 + jnp.exp(-x))
    s = jnp.clip(s, CLIP_LO, CLIP_HI)
    negh = jnp.full((B, C, 1, W), -1.0, s.dtype)
    up = jnp.concatenate([s[:, :, 1:, :], negh], axis=2)
    dn = jnp.concatenate([negh, s[:, :, :-1, :]], axis=2)
    m = jnp.maximum(jnp.maximum(s, up), dn)
    negw = jnp.full((B, C, H, 1), -1.0, s.dtype)
    lf = jnp.concatenate([m[:, :, :, 1:], negw], axis=3)
    rt = jnp.concatenate([negw, m[:, :, :, :-1]], axis=3)
    m = jnp.maximum(jnp.maximum(m, lf), rt)
    o_ref[...] = jnp.where(m == s, s, 0.0)


_nms_call = pl.pallas_call(
    _nms_body,
    out_shape=jax.ShapeDtypeStruct((B, C, H, W), jnp.float32),
    grid=(1,),
    in_specs=[pl.BlockSpec((B, C, H, W), lambda i: (0, 0, 0, 0))],
    out_specs=pl.BlockSpec((B, C, H, W), lambda i: (0, 0, 0, 0)),
)


def _sigmoid_clip(x):
    s = 1.0 / (1.0 + jnp.exp(-x))
    return jnp.clip(s, CLIP_LO, CLIP_HI)


def _sc_body(scores_hbm, xy_hbm, rot_hbm, z_hbm, dim_hbm, out_hbm,
             arr, gm, lsc, lgi, shsc, shgi, csc, cgi, gidxb, gfeat, det, sem):
    c = lax.axis_index("c")
    s = lax.axis_index("s")
    b = c * (B // 2) + s // NQ
    q = s % NQ
    iota = lax.iota(jnp.int32, 16)
    neg1 = jnp.full((16,), -1.0, jnp.float32)
    big = jnp.full((16,), 1 << 30, jnp.int32)
    lane0 = iota == 0

    pltpu.sync_copy(scores_hbm.at[pl.ds(b * CHW + q * NCHUNK, NCHUNK)], arr)

    for v in range(KPAD // 16):
        lsc[pl.ds(v * 16, 16)] = neg1
        lgi[pl.ds(v * 16, 16)] = jnp.zeros((16,), jnp.int32)

    col_base = iota * COLLEN
    nchunk_m1 = jnp.full((16,), NCHUNK - 1, jnp.int32)
    collen_v = jnp.full((16,), COLLEN, jnp.int32)

    def grp_body(g, M):
        gb = col_base + g * 16
        acc = neg1
        for j in range(16):
            v = plsc.load_gather(arr, [gb + j])
            acc = jnp.maximum(acc, v)
        gm[pl.ds(g * 16, 16)] = acc
        return jnp.maximum(M, acc)

    M = lax.fori_loop(0, NGROUPS - 1, grp_body, neg1)
    gb = col_base + (NGROUPS - 1) * 16
    acc = neg1
    for j in range(COLLEN - (NGROUPS - 1) * 16):
        v = plsc.load_gather(arr, [gb + j])
        acc = jnp.maximum(acc, v)
    gm[pl.ds((NGROUPS - 1) * 16, 16)] = acc
    M = jnp.maximum(M, acc)

    def extract_body(i, M):
        m = jnp.max(M)
        mv = jnp.broadcast_to(m, (16,))
        l0v = plsc.all_reduce_ffs(M == mv)

        mink = big
        max2 = neg1
        cnt = jnp.zeros((16,), jnp.int32)
        one_i = jnp.full((16,), 1, jnp.int32)
        zero_i = jnp.zeros((16,), jnp.int32)
        for u in range(NSCAN):
            kk = iota + u * 16
            gidx = jnp.minimum(kk, NGROUPS - 1) * 16 + l0v
            v = plsc.load_gather(gm, [gidx])
            valid = kk < NGROUPS
            hit = (v == mv) & valid
            mink = jnp.minimum(mink, jnp.where(hit, kk, big))
            max2 = jnp.maximum(max2, jnp.where(hit | ~valid, neg1, v))
            cnt = cnt + jnp.where(hit, one_i, zero_i)
        k0 = jnp.min(mink)
        k0v = jnp.broadcast_to(k0, (16,))
        ndup = jnp.sum(cnt)

        tt = k0v * 16 + iota
        eidx = jnp.minimum(l0v * COLLEN + tt, nchunk_m1)
        v = plsc.load_gather(arr, [eidx])
        hit = (v == mv) & (tt < collen_v)
        j0v = plsc.all_reduce_ffs(hit)
        posv = l0v * COLLEN + k0v * 16 + j0v

        iv = jnp.broadcast_to(i, (16,))
        plsc.store_scatter(lsc, [iv], mv, mask=lane0)
        plsc.store_scatter(lgi, [iv], q * NCHUNK + posv, mask=lane0)

        plsc.store_scatter(arr, [posv], neg1, mask=lane0)
        v2 = plsc.load_gather(arr, [eidx])
        v2 = jnp.where(tt < collen_v, v2, neg1)
        gnv = jnp.broadcast_to(jnp.max(v2), (16,))
        plsc.store_scatter(gm, [k0v * 16 + l0v], gnv, mask=lane0)

        cmv = jnp.maximum(jnp.broadcast_to(jnp.max(max2), (16,)), gnv)
        cmv = jnp.where(jnp.broadcast_to(ndup >= 2, (16,)), mv, cmv)
        return jnp.where(iota == l0v, cmv, M)

    lax.fori_loop(0, K, extract_body, M)

    pltpu.sync_copy(lsc, shsc.at[pl.ds(s * KPAD, KPAD)])
    pltpu.sync_copy(lgi, shgi.at[pl.ds(s * KPAD, KPAD)])
    plsc.subcore_barrier()

    @pl.when(q == 0)
    def _leader():
        for j in range(NQ):
            pltpu.sync_copy(shsc.at[pl.ds((s + j) * KPAD, KPAD)],
                            csc.at[pl.ds(j * KPAD, KPAD)])
            pltpu.sync_copy(shgi.at[pl.ds((s + j) * KPAD, KPAD)],
                            cgi.at[pl.ds(j * KPAD, KPAD)])

        def merge_body(i, heads):
            hpos = iota * KPAD + heads
            v = plsc.load_gather(csc, [jnp.minimum(hpos, NQ * KPAD - 1)])
            v = jnp.where(iota < jnp.full((16,), NQ, jnp.int32), v, neg1)
            mx = jnp.max(v)
            mxv = jnp.broadcast_to(mx, (16,))
            j0 = jnp.min(jnp.where(v == mxv, iota, big))
            j0v = jnp.broadcast_to(j0, (16,))
            h0 = jnp.min(jnp.where(iota == j0v, heads, big))
            posv = j0v * KPAD + jnp.broadcast_to(h0, (16,))

            gv = plsc.load_gather(cgi, [posv])
            iv = jnp.broadcast_to(i, (16,))
            plsc.store_scatter(lsc, [iv], mxv, mask=lane0)
            plsc.store_scatter(lgi, [iv], gv, mask=lane0)
            return jnp.where(iota == j0v, heads + 1, heads)

        lax.fori_loop(0, K, merge_body, jnp.zeros((16,), jnp.int32))

        srcs = [(xy_hbm, 2, 0), (xy_hbm, 2, 1), (rot_hbm, 2, 0), (rot_hbm, 2, 1),
                (z_hbm, 1, 0), (dim_hbm, 3, 0), (dim_hbm, 3, 1), (dim_hbm, 3, 2)]
        for v in range(KPAD // 16):
            g_v = lgi[pl.ds(v * 16, 16)]
            hw_v = g_v % HW
            for ch, (_, nch, cin) in enumerate(srcs):
                gidxb[pl.ds(ch * KPAD + v * 16, 16)] = (b * nch + cin) * HW + hw_v

        copies = [pltpu.async_copy(ref.at[gidxb.at[pl.ds(ch * KPAD, KPAD)]],
                                   gfeat.at[pl.ds(ch * KPAD, KPAD)], sem)
                  for ch, (ref, _, _) in enumerate(srcs)]
        for cp in copies:
            cp.wait()

        for v in range(KPAD // 16):
            lanes = iota + v * 16
            ok = lanes < K
            sc_v = lsc[pl.ds(v * 16, 16)]
            g_v = lgi[pl.ds(v * 16, 16)]
            cls_v = (g_v // HW).astype(jnp.float32)
            hw_v = g_v % HW
            row_v = (hw_v // W).astype(jnp.float32)
            col_v = (hw_v % W).astype(jnp.float32)
            offx = _sigmoid_clip(gfeat[pl.ds(0 * KPAD + v * 16, 16)])
            offy = _sigmoid_clip(gfeat[pl.ds(1 * KPAD + v * 16, 16)])
            cols = [sc_v, col_v + offx, row_v + offy,
                    gfeat[pl.ds(4 * KPAD + v * 16, 16)],
                    gfeat[pl.ds(5 * KPAD + v * 16, 16)],
                    gfeat[pl.ds(6 * KPAD + v * 16, 16)],
                    gfeat[pl.ds(7 * KPAD + v * 16, 16)],
                    gfeat[pl.ds(2 * KPAD + v * 16, 16)],
                    gfeat[pl.ds(3 * KPAD + v * 16, 16)],
                    cls_v]
            for cc, val in enumerate(cols):
                plsc.store_scatter(det, [lanes * 10 + cc], val, mask=ok)

        pltpu.sync_copy(det, out_hbm.at[pl.ds(b * 512, 512)])


_sc_call = functools.partial(
    pl.kernel,
    out_type=jax.ShapeDtypeStruct((B * 512,), jnp.float32),
    mesh=plsc.VectorSubcoreMesh(core_axis_name="c", subcore_axis_name="s",
                                num_cores=2, num_subcores=16),
    compiler_params=pltpu.CompilerParams(needs_layout_passes=False),
    scratch_types=[
        pltpu.VMEM((NCHUNK,), jnp.float32),
        pltpu.VMEM((NGROUPS * 16,), jnp.float32),
        pltpu.VMEM((KPAD,), jnp.float32),
        pltpu.VMEM((KPAD,), jnp.int32),
        pltpu.VMEM_SHARED((16 * KPAD,), jnp.float32),
        pltpu.VMEM_SHARED((16 * KPAD,), jnp.int32),
        pltpu.VMEM((NQ * KPAD,), jnp.float32),
        pltpu.VMEM((NQ * KPAD,), jnp.int32),
        pltpu.VMEM((NCH * KPAD,), jnp.int32),
        pltpu.VMEM((NCH * KPAD,), jnp.float32),
        pltpu.VMEM((512,), jnp.float32),
        pltpu.SemaphoreType.DMA,
    ],
)(_sc_body)


def kernel(cls_preds, xy_preds, rot_preds, z_preds, dim_preds, K_arg):
    scores = _nms_call(cls_preds).reshape(B * CHW)
    det = _sc_call(scores, xy_preds.reshape(-1), rot_preds.reshape(-1),
                   z_preds.reshape(-1), dim_preds.reshape(-1))
    out = det.reshape(B, 512)[:, :K * 10].reshape(B, K, 10)
    one = jnp.asarray(K_arg // K_arg, dtype=out.dtype)
    return out * one
